# baseline (speedup 1.0000x reference)
import functools

import jax
import jax.numpy as jnp
from jax import lax
from jax.experimental import pallas as pl
from jax.experimental.pallas import tpu as pltpu
from jax.experimental.pallas import tpu_sc as plsc

N = 50000
E = 800000
F_IN = 44
H = 128
B = 64
SF = 33

BLK = 1024
NBLK = 49
NP = NBLK * BLK
NSUB = 16
NCORE = 2
CHUNK = 128
SUP = 2
EP = 819200
ROWS_SUB = NP // NSUB
NSUP = EP // (CHUNK * SUP)
W = 64


@functools.cache
def _make_sc_agg(mode):
  mesh = plsc.VectorSubcoreMesh(core_axis_name="c", subcore_axis_name="s",
                                num_cores=NCORE, num_subcores=NSUB)
  nsup_w = NSUP // (NCORE * NSUB) if mode == "edges" else NSUP // NSUB

  @functools.partial(
      pl.kernel,
      out_type=jax.ShapeDtypeStruct((2, NP, W), jnp.bfloat16),
      mesh=mesh,
      compiler_params=pltpu.CompilerParams(use_tc_tiling_on_sc=False),
      scratch_types=[
          pltpu.VMEM_SHARED((NP, W), jnp.bfloat16),
          pltpu.VMEM((2, SUP, CHUNK), jnp.int32),
          pltpu.VMEM((2, SUP, CHUNK), jnp.int32),
          pltpu.VMEM((SUP, CHUNK, W), jnp.bfloat16),
          pltpu.VMEM((SUP, CHUNK, W), jnp.bfloat16),
          pltpu.SemaphoreType.DMA,
          pltpu.SemaphoreType.DMA,
          pltpu.SemaphoreType.DMA,
          pltpu.SemaphoreType.DMA,
          pltpu.SemaphoreType.DMA,
          pltpu.SemaphoreType.DMA,
      ],
  )
  def agg(esup_hbm, table_hbm, zeros_hbm, out_hbm,
          acc, idx_a, idx_b, rows_a, rows_b,
          gsem_a, gsem_b, isem_a, isem_b, ssem_a, ssem_b):
    cid = lax.axis_index("c")
    sid = lax.axis_index("s")
    row0 = sid * ROWS_SUB
    if mode == "edges":
      sup0 = (cid * NSUB + sid) * nsup_w
      table_p = table_hbm
    else:
      sup0 = sid * nsup_w
      table_p = table_hbm.at[cid]
    sup_end = sup0 + nsup_w

    pltpu.sync_copy(zeros_hbm, acc.at[pl.ds(row0, ROWS_SUB)])
    plsc.subcore_barrier()

    def fire_gathers(idx_v, rows_v, gsem):
      for b in range(SUP):
        pltpu.async_copy(table_p.at[idx_v.at[0, b]], rows_v.at[b], gsem)

    def drain_gather_fire_scatter(idx_v, rows_v, gsem, ssem):
      for b in range(SUP):
        pltpu.make_async_copy(
            table_p.at[idx_v.at[0, b]], rows_v.at[b], gsem).wait()
        pltpu.async_copy(rows_v.at[b], acc.at[idx_v.at[1, b]], ssem,
                         add=True)

    def drain_scatters(idx_v, rows_v, ssem):
      for b in range(SUP):
        pltpu.make_async_copy(rows_v.at[b], acc.at[idx_v.at[1, b]],
                              ssem).wait()

    def step(g, idx_c, rows_c, gsem_c, isem_c, ssem_c,
             idx_n, rows_n, gsem_n, isem_n, ssem_n):
      @pl.when(g + 1 < sup_end)
      def _():
        @pl.when(g > sup0)
        def _():
          drain_scatters(idx_n, rows_n, ssem_n)
        pltpu.make_async_copy(esup_hbm.at[g + 1], idx_n, isem_n).wait()
        fire_gathers(idx_n, rows_n, gsem_n)
      drain_gather_fire_scatter(idx_c, rows_c, gsem_c, ssem_c)
      @pl.when(g + 2 < sup_end)
      def _():
        pltpu.async_copy(esup_hbm.at[g + 2], idx_c, isem_c)

    pltpu.sync_copy(esup_hbm.at[sup0], idx_a)
    fire_gathers(idx_a, rows_a, gsem_a)
    pltpu.async_copy(esup_hbm.at[sup0 + 1], idx_b, isem_b)

    def body(gg, _):
      g0 = sup0 + 2 * gg
      step(g0, idx_a, rows_a, gsem_a, isem_a, ssem_a,
           idx_b, rows_b, gsem_b, isem_b, ssem_b)
      step(g0 + 1, idx_b, rows_b, gsem_b, isem_b, ssem_b,
           idx_a, rows_a, gsem_a, isem_a, ssem_a)
      return _

    lax.fori_loop(0, nsup_w // 2, body, None)
    drain_scatters(idx_a, rows_a, ssem_a)
    drain_scatters(idx_b, rows_b, ssem_b)
    plsc.subcore_barrier()
    pltpu.sync_copy(acc.at[pl.ds(row0, ROWS_SUB)],
                    out_hbm.at[cid].at[pl.ds(row0, ROWS_SUB)])

  return agg


def _xpanels_body(xt_ref, p_ref, e_ref, o_ref):
  xb = lax.dot_general(xt_ref[...], p_ref[...], (((0,), (0,)), ((), ())),
                       preferred_element_type=jnp.float32)
  o_ref[...] = (xb + e_ref[...]).astype(jnp.bfloat16)


def _xpanels(xt, pmat, e44):
  return pl.pallas_call(
      _xpanels_body,
      grid=(NBLK,),
      in_specs=[
          pl.BlockSpec((F_IN, BLK), lambda i: (0, i)),
          pl.BlockSpec((F_IN, W), lambda i: (0, 0)),
          pl.BlockSpec((1, W), lambda i: (0, 0)),
      ],
      out_specs=pl.BlockSpec((BLK, W), lambda i: (i, 0)),
      out_shape=jax.ShapeDtypeStruct((NP, W), jnp.bfloat16),
  )(xt, pmat, e44)


def _l1_body(agg_ref, xt_ref, wl_ref, bl_ref, wr_ref, o_ref, d_ref):
  a = agg_ref[0].astype(jnp.float32) + agg_ref[1].astype(jnp.float32)
  deg = jnp.maximum(a[:, F_IN:F_IN + 1], 1.0)
  out = jnp.dot(a, wl_ref[...], preferred_element_type=jnp.float32)
  xw = lax.dot_general(xt_ref[...], wr_ref[...], (((0,), (0,)), ((), ())),
                       preferred_element_type=jnp.float32)
  out = out / deg + bl_ref[...] + xw
  nrm = jnp.sqrt(jnp.sum(out * out, axis=1, keepdims=True))
  out = jnp.maximum(out / jnp.maximum(nrm, 1e-12), 0.0).astype(jnp.bfloat16)
  for p in range(2):
    o_ref[p] = out[:, W * p:W * (p + 1)]
  d_ref[...] = deg


def _l1(agg1, x, wl1p, bl1, wr1t):
  return pl.pallas_call(
      _l1_body,
      grid=(NBLK,),
      in_specs=[
          pl.BlockSpec((2, BLK, W), lambda i: (0, i, 0)),
          pl.BlockSpec((F_IN, BLK), lambda i: (0, i)),
          pl.BlockSpec((W, H), lambda i: (0, 0)),
          pl.BlockSpec((1, H), lambda i: (0, 0)),
          pl.BlockSpec((F_IN, H), lambda i: (0, 0)),
      ],
      out_specs=[
          pl.BlockSpec((2, BLK, W), lambda i: (0, i, 0)),
          pl.BlockSpec((BLK, 1), lambda i: (i, 0)),
      ],
      out_shape=[
          jax.ShapeDtypeStruct((2, NP, W), jnp.bfloat16),
          jax.ShapeDtypeStruct((NP, 1), jnp.float32),
      ],
  )(agg1, x, wl1p, bl1, wr1t)


def _l2r_body(agg_ref, hp_ref, d_ref, seg_ref, lo_ref, hi_ref,
              wl_ref, bl_ref, wr_ref,
              data_ref, wsf_ref, bsf_ref, wf1_ref, bf1_ref,
              wf2_ref, bf2_ref, wp_ref, bp_ref, o_ref, acc_ref):
  i = pl.program_id(0)

  @pl.when(i == 0)
  def _():
    acc_ref[...] = jnp.full((B, H), -jnp.inf, jnp.float32)

  deg = d_ref[...]
  aggsum = jnp.zeros((BLK, H), jnp.float32)
  xsum = jnp.zeros((BLK, H), jnp.float32)
  for p in range(2):
    aggsum += jnp.dot(agg_ref[p].astype(jnp.float32),
                      wl_ref[W * p:W * (p + 1), :],
                      preferred_element_type=jnp.float32)
    xsum += jnp.dot(hp_ref[p].astype(jnp.float32),
                    wr_ref[W * p:W * (p + 1), :],
                    preferred_element_type=jnp.float32)
  out = aggsum / deg + bl_ref[...] + xsum
  nrm = jnp.sqrt(jnp.sum(out * out, axis=1, keepdims=True))
  hb = jnp.maximum(out / jnp.maximum(nrm, 1e-12), 0.0)

  lo = lo_ref[0, 0, 0]
  hi = hi_ref[0, 0, 0]
  gidx = i * BLK + lax.broadcasted_iota(jnp.int32, (BLK, 1), 0)
  for s in range(B):
    @pl.when(jnp.logical_and(lo <= s, s <= hi))
    def _():
      mask = jnp.logical_and(gidx >= seg_ref[0, s], gidx < seg_ref[0, s + 1])
      vals = jnp.where(mask, hb, -jnp.inf)
      acc_ref[s:s + 1] = jnp.maximum(acc_ref[s:s + 1],
                                     jnp.max(vals, axis=0, keepdims=True))

  @pl.when(i == NBLK - 1)
  def _():
    g = acc_ref[...]
    g = jnp.where(jnp.isfinite(g), g, 0.0)
    sfeat = jnp.maximum(
        jnp.dot(data_ref[...], wsf_ref[...],
                preferred_element_type=jnp.float32) + bsf_ref[...], 0.0)
    z = jnp.concatenate([g, sfeat], axis=1)
    z = jnp.maximum(
        jnp.dot(z, wf1_ref[...], preferred_element_type=jnp.float32)
        + bf1_ref[...], 0.0)
    z = jnp.maximum(
        jnp.dot(z, wf2_ref[...], preferred_element_type=jnp.float32)
        + bf2_ref[...], 0.0)
    o_ref[...] = jnp.dot(z, wp_ref[...],
                         preferred_element_type=jnp.float32) + bp_ref[...]


def _l2r(agg2, h1p, degv, seg, blo, bhi, wl2t, bl2, wr2t,
         data, wsft, bsf, wf1t, bf1, wf2t, bf2, wpt, bp):
  return pl.pallas_call(
      _l2r_body,
      grid=(NBLK,),
      in_specs=[
          pl.BlockSpec((2, BLK, W), lambda i: (0, i, 0)),
          pl.BlockSpec((2, BLK, W), lambda i: (0, i, 0)),
          pl.BlockSpec((BLK, 1), lambda i: (i, 0)),
          pl.BlockSpec((1, B + 1), lambda i: (0, 0)),
          pl.BlockSpec((1, 1, 1), lambda i: (i, 0, 0)),
          pl.BlockSpec((1, 1, 1), lambda i: (i, 0, 0)),
          pl.BlockSpec((H, H), lambda i: (0, 0)),
          pl.BlockSpec((1, H), lambda i: (0, 0)),
          pl.BlockSpec((H, H), lambda i: (0, 0)),
          pl.BlockSpec((B, SF), lambda i: (0, 0)),
          pl.BlockSpec((SF, H), lambda i: (0, 0)),
          pl.BlockSpec((1, H), lambda i: (0, 0)),
          pl.BlockSpec((2 * H, H), lambda i: (0, 0)),
          pl.BlockSpec((1, H), lambda i: (0, 0)),
          pl.BlockSpec((H, H), lambda i: (0, 0)),
          pl.BlockSpec((1, H), lambda i: (0, 0)),
          pl.BlockSpec((H, 1), lambda i: (0, 0)),
          pl.BlockSpec((1, 1), lambda i: (0, 0)),
      ],
      out_specs=pl.BlockSpec((B, 1), lambda i: (0, 0)),
      out_shape=jax.ShapeDtypeStruct((B, 1), jnp.float32),
      scratch_shapes=[pltpu.VMEM((B, H), jnp.float32)],
  )(agg2, h1p, degv, seg, blo, bhi, wl2t, bl2, wr2t,
    data, wsft, bsf, wf1t, bf1, wf2t, bf2, wpt, bp)


def kernel(x, edge_index, batch, data, Wl1, bl1, Wr1, Wl2, bl2, Wr2,
           Wsf, bsf, Wf1, bf1, Wf2, bf2, Wp, bp):
  src = edge_index[0]
  dst = edge_index[1]
  srcp = jnp.concatenate([src, jnp.zeros((EP - E,), jnp.int32)])
  dstp = jnp.concatenate([dst, jnp.full((EP - E,), N, jnp.int32)])
  esup = jnp.stack([srcp.reshape(NSUP, SUP, CHUNK),
                    dstp.reshape(NSUP, SUP, CHUNK)], axis=1)
  zeros_blk = jnp.zeros((ROWS_SUB, W), jnp.bfloat16)
  batchp = jnp.concatenate([batch, jnp.full((NP - N,), B, jnp.int32)])
  bm = batchp.reshape(NBLK, BLK)
  blo = bm.min(axis=1).reshape(NBLK, 1, 1)
  bhi = bm.max(axis=1).reshape(NBLK, 1, 1)
  seg = jnp.searchsorted(batch, jnp.arange(B + 1, dtype=jnp.int32),
                         side="left").astype(jnp.int32).reshape(1, B + 1)

  wl1p = jnp.pad(Wl1, ((0, 0), (0, W - F_IN))).T
  pmat = jnp.pad(jnp.eye(F_IN, dtype=jnp.float32), ((0, 0), (0, W - F_IN)))
  e44 = jnp.zeros((1, W), jnp.float32).at[0, F_IN].set(1.0)
  xt = x.T
  xc = _xpanels(xt, pmat, e44)
  agg1 = _make_sc_agg("edges")(esup, xc, zeros_blk)
  h1p, degv = _l1(agg1, xt, wl1p, bl1.reshape(1, H), Wr1.T)
  agg2 = _make_sc_agg("cols")(esup, h1p, zeros_blk)
  return _l2r(agg2, h1p, degv, seg, blo, bhi, Wl2.T, bl2.reshape(1, H), Wr2.T,
              data, Wsf.T, bsf.reshape(1, H), Wf1.T, bf1.reshape(1, H),
              Wf2.T, bf2.reshape(1, H), Wp.T, bp.reshape(1, 1))


# speedup vs baseline: 1.0574x; 1.0574x over previous
import functools

import jax
import jax.numpy as jnp
from jax import lax
from jax.experimental import pallas as pl
from jax.experimental.pallas import tpu as pltpu
from jax.experimental.pallas import tpu_sc as plsc

N = 50000
E = 800000
F_IN = 44
H = 128
B = 64
SF = 33

BLK = 1024
NBLK = 49
NP = NBLK * BLK
NSUB = 16
NCORE = 2
CHUNK = 128
SUP = 2
EP = 819200
ROWS_SUB = NP // NSUB
NSUP = EP // (CHUNK * SUP)
W = 64


@functools.cache
def _make_sc_agg(mode):
  mesh = plsc.VectorSubcoreMesh(core_axis_name="c", subcore_axis_name="s",
                                num_cores=NCORE, num_subcores=NSUB)
  nsup_w = NSUP // (NCORE * NSUB) if mode == "edges" else NSUP // NSUB

  @functools.partial(
      pl.kernel,
      out_type=jax.ShapeDtypeStruct((2, NP, W), jnp.bfloat16),
      mesh=mesh,
      compiler_params=pltpu.CompilerParams(use_tc_tiling_on_sc=False),
      scratch_types=[
          pltpu.VMEM_SHARED((NP, W), jnp.bfloat16),
          pltpu.VMEM((2, SUP, CHUNK), jnp.int32),
          pltpu.VMEM((2, SUP, CHUNK), jnp.int32),
          pltpu.VMEM((SUP, CHUNK, W), jnp.bfloat16),
          pltpu.VMEM((SUP, CHUNK, W), jnp.bfloat16),
          pltpu.SemaphoreType.DMA,
          pltpu.SemaphoreType.DMA,
          pltpu.SemaphoreType.DMA,
          pltpu.SemaphoreType.DMA,
          pltpu.SemaphoreType.DMA,
          pltpu.SemaphoreType.DMA,
      ],
  )
  def agg(esup_hbm, table_hbm, zeros_hbm, out_hbm,
          acc, idx_a, idx_b, rows_a, rows_b,
          gsem_a, gsem_b, isem_a, isem_b, ssem_a, ssem_b):
    cid = lax.axis_index("c")
    sid = lax.axis_index("s")
    row0 = sid * ROWS_SUB
    if mode == "edges":
      sup0 = (cid * NSUB + sid) * nsup_w
      table_p = table_hbm
    else:
      sup0 = sid * nsup_w
      table_p = table_hbm.at[cid]
    sup_end = sup0 + nsup_w

    pltpu.sync_copy(zeros_hbm, acc.at[pl.ds(row0, ROWS_SUB)])
    plsc.subcore_barrier()

    def fire_gathers(idx_v, rows_v, gsem):
      for b in range(SUP):
        pltpu.async_copy(table_p.at[idx_v.at[0, b]], rows_v.at[b], gsem)

    def drain_gather_fire_scatter(idx_v, rows_v, gsem, ssem):
      for b in range(SUP):
        pltpu.make_async_copy(
            table_p.at[idx_v.at[0, b]], rows_v.at[b], gsem).wait()
        pltpu.async_copy(rows_v.at[b], acc.at[idx_v.at[1, b]], ssem,
                         add=True)

    def drain_scatters(idx_v, rows_v, ssem):
      for b in range(SUP):
        pltpu.make_async_copy(rows_v.at[b], acc.at[idx_v.at[1, b]],
                              ssem).wait()

    def step(g, idx_c, rows_c, gsem_c, isem_c, ssem_c,
             idx_n, rows_n, gsem_n, isem_n, ssem_n):
      @pl.when(g + 1 < sup_end)
      def _():
        @pl.when(g > sup0)
        def _():
          drain_scatters(idx_n, rows_n, ssem_n)
        pltpu.make_async_copy(esup_hbm.at[g + 1], idx_n, isem_n).wait()
        fire_gathers(idx_n, rows_n, gsem_n)
      drain_gather_fire_scatter(idx_c, rows_c, gsem_c, ssem_c)
      @pl.when(g + 2 < sup_end)
      def _():
        pltpu.async_copy(esup_hbm.at[g + 2], idx_c, isem_c)

    pltpu.sync_copy(esup_hbm.at[sup0], idx_a)
    fire_gathers(idx_a, rows_a, gsem_a)
    pltpu.async_copy(esup_hbm.at[sup0 + 1], idx_b, isem_b)

    def body(gg, _):
      g0 = sup0 + 2 * gg
      step(g0, idx_a, rows_a, gsem_a, isem_a, ssem_a,
           idx_b, rows_b, gsem_b, isem_b, ssem_b)
      step(g0 + 1, idx_b, rows_b, gsem_b, isem_b, ssem_b,
           idx_a, rows_a, gsem_a, isem_a, ssem_a)
      return _

    lax.fori_loop(0, nsup_w // 2, body, None)
    drain_scatters(idx_a, rows_a, ssem_a)
    drain_scatters(idx_b, rows_b, ssem_b)
    plsc.subcore_barrier()
    pltpu.sync_copy(acc.at[pl.ds(row0, ROWS_SUB)],
                    out_hbm.at[cid].at[pl.ds(row0, ROWS_SUB)])

  return agg


def _xpanels_body(x_ref, o_ref):
  xb = x_ref[...].astype(jnp.bfloat16)
  ones = jnp.ones((BLK, 1), jnp.bfloat16)
  zeros = jnp.zeros((BLK, W - F_IN - 1), jnp.bfloat16)
  o_ref[...] = jnp.concatenate([xb, ones, zeros], axis=1)


def _xpanels(x):
  return pl.pallas_call(
      _xpanels_body,
      grid=(NBLK,),
      in_specs=[pl.BlockSpec((BLK, F_IN), lambda i: (i, 0))],
      out_specs=pl.BlockSpec((BLK, W), lambda i: (i, 0)),
      out_shape=jax.ShapeDtypeStruct((NP, W), jnp.bfloat16),
  )(x)


def _l1_body(agg_ref, x_ref, wl_ref, bl_ref, wr_ref, o_ref, d_ref):
  a = agg_ref[0].astype(jnp.float32) + agg_ref[1].astype(jnp.float32)
  deg = jnp.maximum(a[:, F_IN:F_IN + 1], 1.0)
  out = jnp.dot(a, wl_ref[...], preferred_element_type=jnp.float32)
  out = out / deg + bl_ref[...] + jnp.dot(
      x_ref[...], wr_ref[...], preferred_element_type=jnp.float32)
  nrm = jnp.sqrt(jnp.sum(out * out, axis=1, keepdims=True))
  out = jnp.maximum(out / jnp.maximum(nrm, 1e-12), 0.0).astype(jnp.bfloat16)
  for p in range(2):
    o_ref[p] = out[:, W * p:W * (p + 1)]
  d_ref[...] = deg


def _l1(agg1, x, wl1p, bl1, wr1t):
  return pl.pallas_call(
      _l1_body,
      grid=(NBLK,),
      in_specs=[
          pl.BlockSpec((2, BLK, W), lambda i: (0, i, 0)),
          pl.BlockSpec((BLK, F_IN), lambda i: (i, 0)),
          pl.BlockSpec((W, H), lambda i: (0, 0)),
          pl.BlockSpec((1, H), lambda i: (0, 0)),
          pl.BlockSpec((F_IN, H), lambda i: (0, 0)),
      ],
      out_specs=[
          pl.BlockSpec((2, BLK, W), lambda i: (0, i, 0)),
          pl.BlockSpec((BLK, 1), lambda i: (i, 0)),
      ],
      out_shape=[
          jax.ShapeDtypeStruct((2, NP, W), jnp.bfloat16),
          jax.ShapeDtypeStruct((NP, 1), jnp.float32),
      ],
  )(agg1, x, wl1p, bl1, wr1t)


def _l2r_body(agg_ref, hp_ref, d_ref, b_ref, wl_ref, bl_ref, wr_ref,
              data_ref, wsf_ref, bsf_ref, wf1_ref, bf1_ref,
              wf2_ref, bf2_ref, wp_ref, bp_ref, o_ref, acc_ref):
  i = pl.program_id(0)

  @pl.when(i == 0)
  def _():
    acc_ref[...] = jnp.full((B, H), -jnp.inf, jnp.float32)

  deg = d_ref[...]
  aggsum = jnp.zeros((BLK, H), jnp.float32)
  xsum = jnp.zeros((BLK, H), jnp.float32)
  for p in range(2):
    aggsum += jnp.dot(agg_ref[p].astype(jnp.float32),
                      wl_ref[W * p:W * (p + 1), :],
                      preferred_element_type=jnp.float32)
    xsum += jnp.dot(hp_ref[p].astype(jnp.float32),
                    wr_ref[W * p:W * (p + 1), :],
                    preferred_element_type=jnp.float32)
  out = aggsum / deg + bl_ref[...] + xsum
  nrm = jnp.sqrt(jnp.sum(out * out, axis=1, keepdims=True))
  hb = jnp.maximum(out / jnp.maximum(nrm, 1e-12), 0.0)

  bb = b_ref[...]
  lo = jnp.min(bb)
  hi = jnp.max(bb)
  for s in range(B):
    @pl.when(jnp.logical_and(lo <= s, s <= hi))
    def _():
      vals = jnp.where(bb == s, hb, -jnp.inf)
      acc_ref[s:s + 1] = jnp.maximum(acc_ref[s:s + 1],
                                     jnp.max(vals, axis=0, keepdims=True))

  @pl.when(i == NBLK - 1)
  def _():
    g = acc_ref[...]
    g = jnp.where(jnp.isfinite(g), g, 0.0)
    sfeat = jnp.maximum(
        jnp.dot(data_ref[...], wsf_ref[...],
                preferred_element_type=jnp.float32) + bsf_ref[...], 0.0)
    z = jnp.concatenate([g, sfeat], axis=1)
    z = jnp.maximum(
        jnp.dot(z, wf1_ref[...], preferred_element_type=jnp.float32)
        + bf1_ref[...], 0.0)
    z = jnp.maximum(
        jnp.dot(z, wf2_ref[...], preferred_element_type=jnp.float32)
        + bf2_ref[...], 0.0)
    o_ref[...] = jnp.dot(z, wp_ref[...],
                         preferred_element_type=jnp.float32) + bp_ref[...]


def _l2r(agg2, h1p, degv, batchp, wl2t, bl2, wr2t,
         data, wsft, bsf, wf1t, bf1, wf2t, bf2, wpt, bp):
  return pl.pallas_call(
      _l2r_body,
      grid=(NBLK,),
      in_specs=[
          pl.BlockSpec((2, BLK, W), lambda i: (0, i, 0)),
          pl.BlockSpec((2, BLK, W), lambda i: (0, i, 0)),
          pl.BlockSpec((BLK, 1), lambda i: (i, 0)),
          pl.BlockSpec((BLK, 1), lambda i: (i, 0)),
          pl.BlockSpec((H, H), lambda i: (0, 0)),
          pl.BlockSpec((1, H), lambda i: (0, 0)),
          pl.BlockSpec((H, H), lambda i: (0, 0)),
          pl.BlockSpec((B, SF), lambda i: (0, 0)),
          pl.BlockSpec((SF, H), lambda i: (0, 0)),
          pl.BlockSpec((1, H), lambda i: (0, 0)),
          pl.BlockSpec((2 * H, H), lambda i: (0, 0)),
          pl.BlockSpec((1, H), lambda i: (0, 0)),
          pl.BlockSpec((H, H), lambda i: (0, 0)),
          pl.BlockSpec((1, H), lambda i: (0, 0)),
          pl.BlockSpec((H, 1), lambda i: (0, 0)),
          pl.BlockSpec((1, 1), lambda i: (0, 0)),
      ],
      out_specs=pl.BlockSpec((B, 1), lambda i: (0, 0)),
      out_shape=jax.ShapeDtypeStruct((B, 1), jnp.float32),
      scratch_shapes=[pltpu.VMEM((B, H), jnp.float32)],
  )(agg2, h1p, degv, batchp, wl2t, bl2, wr2t,
    data, wsft, bsf, wf1t, bf1, wf2t, bf2, wpt, bp)


def kernel(x, edge_index, batch, data, Wl1, bl1, Wr1, Wl2, bl2, Wr2,
           Wsf, bsf, Wf1, bf1, Wf2, bf2, Wp, bp):
  src = edge_index[0]
  dst = edge_index[1]
  srcp = jnp.concatenate([src, jnp.zeros((EP - E,), jnp.int32)])
  dstp = jnp.concatenate([dst, jnp.full((EP - E,), N, jnp.int32)])
  esup = jnp.stack([srcp.reshape(NSUP, SUP, CHUNK),
                    dstp.reshape(NSUP, SUP, CHUNK)], axis=1)
  zeros_blk = jnp.zeros((ROWS_SUB, W), jnp.bfloat16)
  batchp = jnp.concatenate([batch, jnp.full((NP - N,), B, jnp.int32)])
  batchp = batchp.reshape(NP, 1)

  wl1p = jnp.pad(Wl1, ((0, 0), (0, W - F_IN))).T
  xc = _xpanels(x)
  agg1 = _make_sc_agg("edges")(esup, xc, zeros_blk)
  h1p, degv = _l1(agg1, x, wl1p, bl1.reshape(1, H), Wr1.T)
  agg2 = _make_sc_agg("cols")(esup, h1p, zeros_blk)
  return _l2r(agg2, h1p, degv, batchp, Wl2.T, bl2.reshape(1, H), Wr2.T,
              data, Wsf.T, bsf.reshape(1, H), Wf1.T, bf1.reshape(1, H),
              Wf2.T, bf2.reshape(1, H), Wp.T, bp.reshape(1, 1))
